# baseline (speedup 1.0000x reference)
import jax
import jax.numpy as jnp
from jax.experimental import pallas as pl

_SB = 1


def _spec_kernel(t_ref, f_ref, e_ref, x_ref, o_ref):
    e = e_ref[...]
    for i in range(_SB):
        t = t_ref[i]
        f = f_ref[i]
        o_ref[i] = jnp.where(f, jnp.float32(0.0), jnp.where(t, e, x_ref[i]))


def kernel(x, masked_spec_embed, mask_time, flip_mask, mask_feature):
    B, L, D = x.shape
    t = (mask_time & flip_mask).reshape(B, L, 1)
    f = mask_feature.reshape(B, 1, D)
    e = masked_spec_embed.reshape(1, D).astype(x.dtype)

    grid = (B // _SB,)
    return pl.pallas_call(
        _spec_kernel,
        grid=grid,
        in_specs=[
            pl.BlockSpec((_SB, L, 1), lambda b: (b, 0, 0)),
            pl.BlockSpec((_SB, 1, D), lambda b: (b, 0, 0)),
            pl.BlockSpec((1, D), lambda b: (0, 0)),
            pl.BlockSpec((_SB, L, D), lambda b: (b, 0, 0)),
        ],
        out_specs=pl.BlockSpec((_SB, L, D), lambda b: (b, 0, 0)),
        out_shape=jax.ShapeDtypeStruct((B, L, D), x.dtype),
    )(t, f, e, x)


# speedup vs baseline: 1.0672x; 1.0672x over previous
import functools

import jax
import jax.numpy as jnp
from jax.experimental import pallas as pl
from jax.experimental.pallas import tpu as pltpu

_N = 4


def _spec_kernel(t_hbm, f_hbm, e_ref, x_hbm, o_hbm,
                 tbuf, fbuf, xbuf, obuf, tf_sem, in_sems, out_sems):
    B, L, D = x_hbm.shape

    def in_copy(i, s):
        return pltpu.make_async_copy(x_hbm.at[i], xbuf.at[s], in_sems.at[s])

    def out_copy(i, s):
        return pltpu.make_async_copy(obuf.at[s], o_hbm.at[i], out_sems.at[s])

    mt = pltpu.make_async_copy(t_hbm, tbuf, tf_sem)
    mf = pltpu.make_async_copy(f_hbm, fbuf, tf_sem)
    mt.start()
    mf.start()
    for s in range(_N):
        in_copy(s, s).start()
    mt.wait()
    mf.wait()

    e = e_ref[...]

    def step(i, carry):
        s = jax.lax.rem(i, _N)
        in_copy(i, s).wait()

        @pl.when(i >= _N)
        def _():
            out_copy(i - _N, s).wait()

        t = tbuf[i] != 0
        f = fbuf[i] != 0
        obuf[s] = jnp.where(f, jnp.float32(0.0), jnp.where(t, e, xbuf[s]))
        out_copy(i, s).start()

        @pl.when(i + _N < B)
        def _():
            in_copy(i + _N, s).start()

        return carry

    jax.lax.fori_loop(0, B, step, 0)
    for s in range(_N):
        out_copy(B - _N + s, jax.lax.rem(B - _N + s, _N)).wait()


def kernel(x, masked_spec_embed, mask_time, flip_mask, mask_feature):
    B, L, D = x.shape
    t = (mask_time & flip_mask).astype(jnp.int8).reshape(B, L, 1)
    f = mask_feature.astype(jnp.int8).reshape(B, 1, D)
    e = masked_spec_embed.reshape(1, D).astype(x.dtype)

    return pl.pallas_call(
        functools.partial(_spec_kernel),
        in_specs=[
            pl.BlockSpec(memory_space=pl.ANY),
            pl.BlockSpec(memory_space=pl.ANY),
            pl.BlockSpec((1, D), lambda: (0, 0)),
            pl.BlockSpec(memory_space=pl.ANY),
        ],
        out_specs=pl.BlockSpec(memory_space=pl.ANY),
        out_shape=jax.ShapeDtypeStruct((B, L, D), x.dtype),
        scratch_shapes=[
            pltpu.VMEM((B, L, 1), jnp.int8),
            pltpu.VMEM((B, 1, D), jnp.int8),
            pltpu.VMEM((_N, L, D), x.dtype),
            pltpu.VMEM((_N, L, D), x.dtype),
            pltpu.SemaphoreType.DMA,
            pltpu.SemaphoreType.DMA((_N,)),
            pltpu.SemaphoreType.DMA((_N,)),
        ],
    )(t, f, e, x)


# speedup vs baseline: 1.1465x; 1.0743x over previous
import jax
import jax.numpy as jnp
from jax.experimental import pallas as pl
from jax.experimental.pallas import tpu as pltpu

_N = 6


def _spec_kernel(t_ref, fl_ref, f_ref, e_ref, x_hbm, o_hbm,
                 eye, xbuf, obuf, in_sems, out_sems):
    B, L, D = x_hbm.shape

    def in_copy(i, s):
        return pltpu.make_async_copy(x_hbm.at[i], xbuf.at[s], in_sems.at[s])

    def out_copy(i, s):
        return pltpu.make_async_copy(obuf.at[s], o_hbm.at[i], out_sems.at[s])

    for s in range(_N):
        in_copy(s, s).start()

    rows = jax.lax.broadcasted_iota(jnp.int32, (L, L), 0)
    cols = jax.lax.broadcasted_iota(jnp.int32, (L, L), 1)
    eye[...] = jnp.where(rows == cols, jnp.float32(1), jnp.float32(0))

    e = e_ref[...]

    def step(i, carry):
        s = jax.lax.rem(i, _N)

        trow = jnp.where(jnp.logical_and(t_ref[pl.ds(i, 1)] != 0,
                                         fl_ref[pl.ds(i, 1)] != 0),
                         jnp.float32(1), jnp.float32(0))
        tcol = jax.lax.dot_general(
            eye[...], trow, (((1,), (1,)), ((), ())),
            preferred_element_type=jnp.float32)
        t = tcol != 0.0
        f = f_ref[pl.ds(i, 1)] != 0

        in_copy(i, s).wait()

        @pl.when(i >= _N)
        def _():
            out_copy(i - _N, s).wait()

        obuf[s] = jnp.where(f, jnp.float32(0.0), jnp.where(t, e, xbuf[s]))
        out_copy(i, s).start()

        @pl.when(i + _N < B)
        def _():
            in_copy(i + _N, s).start()

        return carry

    jax.lax.fori_loop(0, B, step, 0)
    for s in range(_N):
        out_copy(B - _N + s, jax.lax.rem(B - _N + s, _N)).wait()


def kernel(x, masked_spec_embed, mask_time, flip_mask, mask_feature):
    B, L, D = x.shape
    e = masked_spec_embed.reshape(1, D).astype(x.dtype)

    return pl.pallas_call(
        _spec_kernel,
        in_specs=[
            pl.BlockSpec((B, L), lambda: (0, 0)),
            pl.BlockSpec((B, L), lambda: (0, 0)),
            pl.BlockSpec((B, D), lambda: (0, 0)),
            pl.BlockSpec((1, D), lambda: (0, 0)),
            pl.BlockSpec(memory_space=pl.ANY),
        ],
        out_specs=pl.BlockSpec(memory_space=pl.ANY),
        out_shape=jax.ShapeDtypeStruct((B, L, D), x.dtype),
        scratch_shapes=[
            pltpu.VMEM((L, L), jnp.float32),
            pltpu.VMEM((_N, L, D), x.dtype),
            pltpu.VMEM((_N, L, D), x.dtype),
            pltpu.SemaphoreType.DMA((_N,)),
            pltpu.SemaphoreType.DMA((_N,)),
        ],
    )(mask_time, flip_mask, mask_feature, e, x)
